# Initial kernel scaffold; baseline (speedup 1.0000x reference)
import jax
import jax.numpy as jnp
from jax.experimental import pallas as pl


def kernel(predictions, targets):
    raise NotImplementedError("write your pallas kernel here")


# speedup vs baseline: 18.9498x; 18.9498x over previous
import functools

import jax
import jax.numpy as jnp
from jax.experimental import pallas as pl
from jax.experimental.pallas import tpu as pltpu


def _trim_body(p_ref, t_ref, o_ref, r_ref, *, m):
    r = jnp.abs(p_ref[...] - t_ref[...])
    r_ref[...] = r

    def step(_, lohi):
        lo, hi = lohi
        mid = lo + (hi - lo) // 2
        midf = jax.lax.bitcast_convert_type(mid, jnp.float32)
        c = jnp.sum((r_ref[...] <= midf).astype(jnp.int32))
        keep = c >= m
        return (jnp.where(keep, lo, mid + 1), jnp.where(keep, mid, hi))

    lo, _ = jax.lax.fori_loop(0, 31, step, (jnp.int32(0), jnp.int32(0x7F800000)))
    t = jax.lax.bitcast_convert_type(lo, jnp.float32)
    rr = r_ref[...]
    less = rr < t
    c_less = jnp.sum(less.astype(jnp.int32))
    s_less = jnp.sum(jnp.where(less, rr, 0.0))
    o_ref[0, 0] = (s_less + t * (m - c_less).astype(jnp.float32)) / jnp.float32(m)


def kernel(predictions, targets):
    n = predictions.size
    k = int(0.1 * n)
    m = n - k
    out = pl.pallas_call(
        functools.partial(_trim_body, m=m),
        out_shape=jax.ShapeDtypeStruct((1, 1), jnp.float32),
        out_specs=pl.BlockSpec(memory_space=pltpu.SMEM),
        scratch_shapes=[pltpu.VMEM(predictions.shape, jnp.float32)],
    )(predictions, targets)
    return out[0, 0]
